# baseline (speedup 1.0000x reference)
import functools

import jax
import jax.numpy as jnp
from jax import lax
from jax.experimental import pallas as pl
from jax.experimental.pallas import tpu as pltpu
from jax.experimental.pallas import tpu_sc as plsc

_ROWS = 32768
_K = 32
_N = 8192
_HALF = _N // 2
_TILE = 1024
_NT = _ROWS // _TILE

_NW = 32
_BPW = _ROWS // _NW
_CH = 128
_NCH = _BPW // _CH


def _dist_argmin_body(z_ref, wm2_ref, s1_ref, w2_ref, idx_ref, dsum_ref):
    i = pl.program_id(0)

    @pl.when(i == 0)
    def _init():
        dsum_ref[...] = jnp.zeros_like(dsum_ref)

    z = z_ref[...]
    mm = lax.dot_general(z, wm2_ref[...], (((1,), (1,)), ((), ())),
                         preferred_element_type=jnp.float32)
    s1 = s1_ref[...]
    RG = 64
    lane = lax.broadcasted_iota(jnp.int32, (RG, 128), 1)
    parts = [[], [], [], []]
    for r in range(0, _TILE, RG):
        s1r = s1[r:r + RG, :]
        for h in range(2):
            vacc = jnp.full((RG, 128), jnp.inf, jnp.float32)
            tacc = jnp.zeros((RG, 128), jnp.int32)
            for t in range(_HALF // 128):
                sl = h * _HALF + t * 128
                chunk = (s1r + mm[r:r + RG, sl:sl + 128]) + w2_ref[:, sl:sl + 128]
                m = chunk < vacc
                vacc = jnp.where(m, chunk, vacc)
                tacc = jnp.where(m, t, tacc)
            v = jnp.min(vacc, axis=1, keepdims=True)
            jfull = (tacc * 128 + lane) + h * _HALF
            a = jnp.min(jnp.where(vacc == v, jfull, _N), axis=1, keepdims=True)
            parts[2 * h].append(v)
            parts[2 * h + 1].append(a)
    v0, a0, v1, a1 = (jnp.concatenate(p, axis=0) for p in parts)
    c = v0.astype(jnp.bfloat16).astype(jnp.float32)
    pick1 = v1 < c
    idx = jnp.where(pick1, a1, a0)
    dsel = jnp.where(pick1, v1, v0)
    idx_ref[...] = idx.reshape(1, 1, _TILE)
    dsum_ref[...] += jnp.sum(dsel).reshape(1, 1)


def _dist_argmin(z_flat, wm2, s1, w2):
    return pl.pallas_call(
        _dist_argmin_body,
        grid=(_NT,),
        in_specs=[
            pl.BlockSpec((_TILE, _K), lambda i: (i, 0)),
            pl.BlockSpec((_N, _K), lambda i: (0, 0)),
            pl.BlockSpec((_TILE, 1), lambda i: (i, 0)),
            pl.BlockSpec((1, _N), lambda i: (0, 0)),
        ],
        out_specs=[
            pl.BlockSpec((1, 1, _TILE), lambda i: (i, 0, 0)),
            pl.BlockSpec((1, 1), lambda i: (0, 0)),
        ],
        out_shape=[
            jax.ShapeDtypeStruct((_NT, 1, _TILE), jnp.int32),
            jax.ShapeDtypeStruct((1, 1), jnp.float32),
        ],
    )(z_flat, wm2, s1, w2)


def _sc_gather(w, idx3):
    mesh = plsc.VectorSubcoreMesh(core_axis_name="c", subcore_axis_name="s")

    @functools.partial(
        pl.kernel,
        mesh=mesh,
        compiler_params=pltpu.CompilerParams(use_tc_tiling_on_sc=False),
        out_type=jax.ShapeDtypeStruct((_ROWS, _K), jnp.float32),
        scratch_types=[
            pltpu.VMEM((_NCH, _CH), jnp.int32),
            pltpu.VMEM((_BPW, _K), jnp.float32),
            pltpu.SemaphoreType.DMA,
        ],
    )
    def k(table_hbm, idx_hbm, out_hbm, idx_v, rows_v, sem):
        wid = lax.axis_index("s") * 2 + lax.axis_index("c")
        pltpu.sync_copy(idx_hbm.at[wid], idx_v)
        copies = [
            pltpu.async_copy(table_hbm.at[idx_v.at[j]],
                             rows_v.at[pl.ds(j * _CH, _CH)], sem)
            for j in range(_NCH)
        ]
        for c in copies:
            c.wait()
        pltpu.sync_copy(rows_v, out_hbm.at[pl.ds(wid * _BPW, _BPW)])

    return k(w, idx3)


def kernel(z, W):
    B, C, H, Wd = z.shape
    z_flat = jnp.transpose(z, (0, 2, 3, 1)).reshape(-1, C)
    s1 = jnp.sum(z_flat ** 2, axis=1, keepdims=True)
    w2 = jnp.sum(W ** 2, axis=1).reshape(1, _N)
    idx3, dsum = _dist_argmin(z_flat, -2.0 * W, s1, w2)
    q_flat = _sc_gather(W, idx3.reshape(_NW, _NCH, _CH))
    q = jnp.transpose(q_flat.reshape(B, H, Wd, C), (0, 3, 1, 2))
    n = B * C * H * Wd
    m = dsum[0, 0]
    loss = m * (jnp.float32(1.0 / n) + jnp.float32(0.25 / n))
    q_st = z + lax.stop_gradient(q - z)
    return (q_st, loss)


# speedup vs baseline: 1.0322x; 1.0322x over previous
import functools

import jax
import jax.numpy as jnp
from jax import lax
from jax.experimental import pallas as pl
from jax.experimental.pallas import tpu as pltpu
from jax.experimental.pallas import tpu_sc as plsc

_ROWS = 32768
_K = 32
_N = 8192
_HALF = _N // 2
_TILE = 1024
_NT = _ROWS // _TILE

_NW = 32
_BPW = _ROWS // _NW
_CH = 128
_NCH = _BPW // _CH


def _dist_argmin_body(z_ref, wm2_ref, s1_ref, w2_ref, idx_ref, dsum_ref):
    i = pl.program_id(0)

    @pl.when(i == 0)
    def _init():
        dsum_ref[...] = jnp.zeros_like(dsum_ref)

    z = z_ref[...]
    mm = lax.dot_general(z, wm2_ref[...], (((1,), (1,)), ((), ())),
                         preferred_element_type=jnp.float32)
    s1 = s1_ref[...]
    RG = 64
    lane = lax.broadcasted_iota(jnp.int32, (RG, 128), 1)
    parts = [[], [], [], []]
    for r in range(0, _TILE, RG):
        s1r = s1[r:r + RG, :]
        for h in range(2):
            vacc = jnp.full((RG, 128), jnp.inf, jnp.float32)
            tacc = jnp.zeros((RG, 128), jnp.int32)
            for t in range(_HALF // 128):
                sl = h * _HALF + t * 128
                chunk = (s1r + mm[r:r + RG, sl:sl + 128]) + w2_ref[:, sl:sl + 128]
                m = chunk < vacc
                vacc = jnp.where(m, chunk, vacc)
                tacc = jnp.where(m, t, tacc)
            v = jnp.min(vacc, axis=1, keepdims=True)
            jfull = (tacc * 128 + lane) + h * _HALF
            a = jnp.min(jnp.where(vacc == v, jfull, _N), axis=1, keepdims=True)
            parts[2 * h].append(v)
            parts[2 * h + 1].append(a)
    v0, a0, v1, a1 = (jnp.concatenate(p, axis=0) for p in parts)
    c = v0.astype(jnp.bfloat16).astype(jnp.float32)
    pick1 = v1 < c
    idx = jnp.where(pick1, a1, a0)
    dsel = jnp.where(pick1, v1, v0)
    idx_ref[...] = idx.reshape(1, 1, _TILE)
    dsum_ref[...] += jnp.sum(dsel).reshape(1, 1)


def _dist_argmin(z_flat, wm2, s1, w2):
    return pl.pallas_call(
        _dist_argmin_body,
        grid=(_NT,),
        in_specs=[
            pl.BlockSpec((_TILE, _K), lambda i: (i, 0)),
            pl.BlockSpec((_N, _K), lambda i: (0, 0)),
            pl.BlockSpec((_TILE, 1), lambda i: (i, 0)),
            pl.BlockSpec((1, _N), lambda i: (0, 0)),
        ],
        out_specs=[
            pl.BlockSpec((1, 1, _TILE), lambda i: (i, 0, 0)),
            pl.BlockSpec((1, 1), lambda i: (0, 0)),
        ],
        out_shape=[
            jax.ShapeDtypeStruct((_NT, 1, _TILE), jnp.int32),
            jax.ShapeDtypeStruct((1, 1), jnp.float32),
        ],
    )(z_flat, wm2, s1, w2)


def _sc_gather(w, idx3):
    mesh = plsc.VectorSubcoreMesh(core_axis_name="c", subcore_axis_name="s")

    @functools.partial(
        pl.kernel,
        mesh=mesh,
        compiler_params=pltpu.CompilerParams(use_tc_tiling_on_sc=False),
        out_type=jax.ShapeDtypeStruct((_ROWS, _K), jnp.float32),
        scratch_types=[
            pltpu.VMEM((_NCH, _CH), jnp.int32),
            pltpu.VMEM((_BPW, _K), jnp.float32),
            pltpu.SemaphoreType.DMA,
        ],
    )
    def k(table_hbm, idx_hbm, out_hbm, idx_v, rows_v, sem):
        wid = lax.axis_index("s") * 2 + lax.axis_index("c")
        pltpu.sync_copy(idx_hbm.at[wid], idx_v)
        copies = [
            pltpu.async_copy(table_hbm.at[idx_v.at[j]],
                             rows_v.at[pl.ds(j * _CH, _CH)], sem)
            for j in range(_NCH)
        ]
        for c in copies:
            c.wait()
        pltpu.sync_copy(rows_v, out_hbm.at[pl.ds(wid * _BPW, _BPW)])

    return k(w, idx3)


def kernel(z, W):
    B, C, H, Wd = z.shape
    z_flat = jnp.transpose(z, (0, 2, 3, 1)).reshape(-1, C)
    s1 = jnp.sum(z_flat ** 2, axis=1, keepdims=True)
    w2 = jnp.sum(W ** 2, axis=1).reshape(1, _N)
    idx3, dsum = _dist_argmin(z_flat.astype(jnp.bfloat16), -2.0 * W, s1, w2)
    q_flat = _sc_gather(W, idx3.reshape(_NW, _NCH, _CH))
    q = jnp.transpose(q_flat.reshape(B, H, Wd, C), (0, 3, 1, 2))
    n = B * C * H * Wd
    m = dsum[0, 0]
    loss = m * (jnp.float32(1.0 / n) + jnp.float32(0.25 / n))
    q_st = z + lax.stop_gradient(q - z)
    return (q_st, loss)
